# baseline (speedup 1.0000x reference)
import jax
import jax.numpy as jnp
from jax import lax
from jax.experimental import pallas as pl
from jax.experimental.pallas import tpu as pltpu
from jax.experimental.pallas import tpu_sc as plsc

B = 16384
L = 200
EMB = 32
N = B * L

_info = plsc.get_sparse_core_info()
NC, NS = _info.num_cores, _info.num_subcores
NW = NC * NS
B_PER_W = N // NW
CHUNK = 512
S = B_PER_W // CHUNK
UNROLL = 16


def _emb_kernel(idx_hbm, table_hbm, out_hbm, *scratch):
    idx_v = scratch[0:2]
    rows_v = scratch[2:4]
    tp_v = scratch[4:6]
    sem_i = scratch[6:8]
    sem_g = scratch[8:10]
    sem_o = scratch[10:12]

    wid = lax.axis_index("s") * NC + lax.axis_index("c")
    base = wid * B_PER_W
    viota = lax.iota(jnp.int32, 16)
    viota_hi = viota + 16

    def idx_copy(c, k):
        return pltpu.make_async_copy(
            idx_hbm.at[pl.ds(base + c * CHUNK, CHUNK)], idx_v[k], sem_i[k])

    def gather_copy(k):
        return pltpu.make_async_copy(table_hbm.at[idx_v[k]], rows_v[k], sem_g[k])

    def wb_copy(c, k):
        flat = base + c * CHUNK
        l = flat // B
        b0 = flat % B
        return pltpu.make_async_copy(
            tp_v[k], out_hbm.at[l, :, pl.ds(b0, CHUNK)], sem_o[k])

    def transpose_chunk(k):
        rows = rows_v[k]
        tp = tp_v[k]

        @plsc.parallel_loop(0, CHUNK, step=1, unroll=UNROLL)
        def _(j):
            lo = rows[j, pl.ds(0, 16)]
            hi = rows[j, pl.ds(16, 16)]
            jv = jnp.full((16,), j, jnp.int32)
            plsc.store_scatter(tp, [viota, jv], lo)
            plsc.store_scatter(tp, [viota_hi, jv], hi)

    idx_copy(0, 0).start()
    idx_copy(1, 1).start()

    def body(j, carry):
        for k in (0, 1):
            c = 2 * j + k

            idx_copy(c, k).wait()
            gather_copy(k).start()

            @pl.when(c >= 1)
            def _():
                ko = 1 - k
                gather_copy(ko).wait()

                @pl.when(c + 1 < S)
                def _():
                    idx_copy(c + 1, ko).start()

                @pl.when(c >= 3)
                def _():
                    wb_copy(c - 3, ko).wait()

                transpose_chunk(ko)
                wb_copy(c - 1, ko).start()
        return carry

    lax.fori_loop(0, S // 2, body, 0)

    kl = (S - 1) % 2
    gather_copy(kl).wait()
    wb_copy(S - 3, kl).wait()
    transpose_chunk(kl)
    wb_copy(S - 1, kl).start()
    wb_copy(S - 2, 1 - kl).wait()
    wb_copy(S - 1, kl).wait()


def kernel(x, table):
    idx = x.T.reshape(N)
    mesh = plsc.VectorSubcoreMesh(core_axis_name="c", subcore_axis_name="s")
    out3 = pl.kernel(
        _emb_kernel,
        mesh=mesh,
        out_type=jax.ShapeDtypeStruct((L, EMB, B), jnp.float32),
        scratch_types=(
            [pltpu.VMEM((CHUNK,), jnp.int32) for _ in range(2)]
            + [pltpu.VMEM((CHUNK, EMB), jnp.float32) for _ in range(2)]
            + [pltpu.VMEM((EMB, CHUNK), jnp.float32) for _ in range(2)]
            + [pltpu.SemaphoreType.DMA for _ in range(6)]
        ),
        compiler_params=pltpu.CompilerParams(use_tc_tiling_on_sc=False, needs_layout_passes=False),
    )(idx, table)
    return jnp.transpose(out3, (2, 0, 1))


# speedup vs baseline: 1.0459x; 1.0459x over previous
import jax
import jax.numpy as jnp
from jax import lax
from jax.experimental import pallas as pl
from jax.experimental.pallas import tpu as pltpu
from jax.experimental.pallas import tpu_sc as plsc

B = 16384
L = 200
EMB = 32
N = B * L

_info = plsc.get_sparse_core_info()
NC, NS = _info.num_cores, _info.num_subcores
NW = NC * NS
B_PER_W = N // NW
CHUNK = 512
S = B_PER_W // CHUNK
UNROLL = 8


def _emb_kernel(idx_hbm, table_hbm, out_hbm, *scratch):
    idx_v = scratch[0:2]
    rows_v = scratch[2:4]
    tp_v = scratch[4:6]
    sem_i = scratch[6:8]
    sem_g = scratch[8:10]
    sem_o = scratch[10:12]

    wid = lax.axis_index("s") * NC + lax.axis_index("c")
    base = wid * B_PER_W
    viota = lax.iota(jnp.int32, 16)
    viota_hi = viota + 16

    def idx_copy(c, k):
        return pltpu.make_async_copy(
            idx_hbm.at[pl.ds(base + c * CHUNK, CHUNK)], idx_v[k], sem_i[k])

    def gather_copy(k):
        return pltpu.make_async_copy(table_hbm.at[idx_v[k]], rows_v[k], sem_g[k])

    def wb_copy(c, k):
        flat = base + c * CHUNK
        l = flat // B
        b0 = flat % B
        return pltpu.make_async_copy(
            tp_v[k], out_hbm.at[l, :, pl.ds(b0, CHUNK)], sem_o[k])

    col_consts = [jnp.full((16,), e, jnp.int32) for e in range(EMB)]

    def transpose_chunk(k):
        rows = rows_v[k]
        tp = tp_v[k]

        @plsc.parallel_loop(0, CHUNK // 16, step=1, unroll=2)
        def _(j0):
            row_idx = viota + j0 * 16
            for e in range(EMB):
                vals = plsc.load_gather(rows, [row_idx, col_consts[e]])
                tp[e, pl.ds(j0 * 16, 16)] = vals

    idx_copy(0, 0).start()
    idx_copy(1, 1).start()

    def body(j, carry):
        for k in (0, 1):
            c = 2 * j + k

            idx_copy(c, k).wait()
            gather_copy(k).start()

            @pl.when(c >= 1)
            def _():
                ko = 1 - k
                gather_copy(ko).wait()

                @pl.when(c + 1 < S)
                def _():
                    idx_copy(c + 1, ko).start()

                @pl.when(c >= 3)
                def _():
                    wb_copy(c - 3, ko).wait()

                transpose_chunk(ko)
                wb_copy(c - 1, ko).start()
        return carry

    lax.fori_loop(0, S // 2, body, 0)

    kl = (S - 1) % 2
    gather_copy(kl).wait()
    wb_copy(S - 3, kl).wait()
    transpose_chunk(kl)
    wb_copy(S - 1, kl).start()
    wb_copy(S - 2, 1 - kl).wait()
    wb_copy(S - 1, kl).wait()


def kernel(x, table):
    idx = x.T.reshape(N)
    mesh = plsc.VectorSubcoreMesh(core_axis_name="c", subcore_axis_name="s")
    out3 = pl.kernel(
        _emb_kernel,
        mesh=mesh,
        out_type=jax.ShapeDtypeStruct((L, EMB, B), jnp.float32),
        scratch_types=(
            [pltpu.VMEM((CHUNK,), jnp.int32) for _ in range(2)]
            + [pltpu.VMEM((CHUNK, EMB), jnp.float32) for _ in range(2)]
            + [pltpu.VMEM((EMB, CHUNK), jnp.float32) for _ in range(2)]
            + [pltpu.SemaphoreType.DMA for _ in range(6)]
        ),
        compiler_params=pltpu.CompilerParams(use_tc_tiling_on_sc=False, needs_layout_passes=False),
    )(idx, table)
    return jnp.transpose(out3, (2, 0, 1))
